# baseline (speedup 1.0000x reference)
import jax
import jax.numpy as jnp
from jax import lax
from jax.experimental import pallas as pl
from jax.experimental.pallas import tpu as pltpu
from jax.experimental.pallas import tpu_sc as plsc

D = 256
K = 64
B = 8
S = 256
ROWS = B * S
FLAT = D * K
NW = 32
RPW = ROWS // NW
RB = 2
LANES = 16


def _gate_topk_body(wt_ref, b_ref, imp_ref, fw_ref, sel_ref):
    imp = imp_ref[...]
    m = jnp.max(imp, axis=1, keepdims=True)
    e = jnp.exp(imp - m)
    wts = e / jnp.sum(e, axis=1, keepdims=True)
    wt = wt_ref[...].astype(jnp.bfloat16).astype(jnp.float32)
    g = jax.nn.sigmoid(wt + b_ref[...])
    fw = wts * g
    fw_ref[...] = fw

    jidx = lax.broadcasted_iota(jnp.int32, (D, D), 1)
    work = fw
    for r in range(K):
        mx = jnp.max(work, axis=1, keepdims=True)
        cand = jnp.where(work == mx, jidx, D)
        j = jnp.min(cand, axis=1, keepdims=True)
        sel_ref[:, r:r + 1] = j
        work = jnp.where(jidx == j, -jnp.inf, work)


_gate_topk = pl.pallas_call(
    _gate_topk_body,
    out_shape=(
        jax.ShapeDtypeStruct((D, D), jnp.float32),
        jax.ShapeDtypeStruct((D, K), jnp.int32),
    ),
)


def _sc_gather_body(x_hbm, idx_hbm, out_hbm,
                    idx_v, xr0, xr1, ov0, ov1, sem0, sem1):
    cid = lax.axis_index("c")
    sid = lax.axis_index("s")
    wid = sid * 2 + cid
    row0 = wid * RPW
    pltpu.sync_copy(idx_hbm, idx_v)

    bufs = ((xr0, ov0, sem0), (xr1, ov1, sem1))

    def pair(i, _):
        for half in range(2):
            xr, ov, sem = bufs[half]
            rb = i * 2 + half
            base = row0 + rb * RB

            dst = out_hbm.at[pl.ds(base * FLAT, RB * FLAT)]

            @pl.when(i > 0)
            def _():
                pltpu.make_async_copy(ov, dst, sem).wait()

            pltpu.sync_copy(x_hbm.at[pl.ds(base * D, RB * D)], xr)

            @plsc.parallel_loop(0, FLAT // LANES, unroll=8)
            def _(o):
                off = o * LANES
                iv = idx_v[pl.ds(off, LANES)]
                for r in range(RB):
                    ov[pl.ds(r * FLAT + off, LANES)] = (
                        plsc.load_gather(xr, [iv + r * D]))

            pltpu.async_copy(ov, dst, sem)
        return 0

    lax.fori_loop(0, RPW // (2 * RB), pair, 0)
    for half in range(2):
        xr, ov, sem = bufs[half]
        pltpu.make_async_copy(ov, out_hbm.at[pl.ds(row0 * FLAT, RB * FLAT)], sem).wait()


_sc_gather_cached = None


def _sc_gather(x_flat, idx_flat):
    global _sc_gather_cached
    if _sc_gather_cached is None:
        _sc_gather_cached = pl.kernel(
            _sc_gather_body,
            out_type=jax.ShapeDtypeStruct((ROWS * FLAT,), jnp.float32),
            mesh=plsc.VectorSubcoreMesh(
                core_axis_name="c", subcore_axis_name="s",
                num_cores=2, num_subcores=16,
            ),
            compiler_params=pltpu.CompilerParams(needs_layout_passes=False),
            scratch_types=[
                pltpu.VMEM((FLAT,), jnp.int32),
                pltpu.VMEM((RB * D,), jnp.float32),
                pltpu.VMEM((RB * D,), jnp.float32),
                pltpu.VMEM((RB * FLAT,), jnp.float32),
                pltpu.VMEM((RB * FLAT,), jnp.float32),
                pltpu.SemaphoreType.DMA,
                pltpu.SemaphoreType.DMA,
            ],
        )
    return _sc_gather_cached(x_flat, idx_flat)


def kernel(x, importance_scores, W, b):
    fw, sel = _gate_topk(W.T, b.reshape(1, D), importance_scores.reshape(1, D))
    out = _sc_gather(x.reshape(ROWS * D), sel.reshape(FLAT))
    return out.reshape(B, S, D, K), fw


# speedup vs baseline: 1.0652x; 1.0652x over previous
import jax
import jax.numpy as jnp
from jax import lax
from jax.experimental import pallas as pl
from jax.experimental.pallas import tpu as pltpu
from jax.experimental.pallas import tpu_sc as plsc

D = 256
K = 64
B = 8
S = 256
ROWS = B * S
FLAT = D * K
NW = 32
RPW = ROWS // NW
RB = 2
LANES = 16


def _gate_topk_body(wt_ref, b_ref, imp_ref, fw_ref, sel_ref):
    imp = imp_ref[...]
    m = jnp.max(imp, axis=1, keepdims=True)
    e = jnp.exp(imp - m)
    wts = e / jnp.sum(e, axis=1, keepdims=True)
    wt = wt_ref[...].astype(jnp.bfloat16).astype(jnp.float32)
    g = jax.nn.sigmoid(wt + b_ref[...])
    fw = wts * g
    fw_ref[...] = fw

    jidx = lax.broadcasted_iota(jnp.int32, (D, D), 1)
    work = fw
    for r in range(K):
        mx = jnp.max(work, axis=1, keepdims=True)
        cand = jnp.where(work == mx, jidx, D)
        j = jnp.min(cand, axis=1, keepdims=True)
        sel_ref[:, r:r + 1] = j
        work = jnp.where(jidx == j, -jnp.inf, work)


_gate_topk = pl.pallas_call(
    _gate_topk_body,
    out_shape=(
        jax.ShapeDtypeStruct((D, D), jnp.float32),
        jax.ShapeDtypeStruct((D, K), jnp.int32),
    ),
)


def _sc_gather_body(x_hbm, idx_hbm, out_hbm,
                    idx_v, xr0, xr1, ov0, ov1, sem0, sem1):
    cid = lax.axis_index("c")
    sid = lax.axis_index("s")
    wid = sid * 2 + cid
    row0 = wid * RPW
    pltpu.sync_copy(idx_hbm, idx_v)

    bufs = ((xr0, ov0, sem0), (xr1, ov1, sem1))

    def pair(i, _):
        for half in range(2):
            xr, ov, sem = bufs[half]
            rb = i * 2 + half
            base = row0 + rb * RB

            dst = out_hbm.at[pl.ds(base * FLAT, RB * FLAT)]

            @pl.when(i > 0)
            def _():
                pltpu.make_async_copy(ov, dst, sem).wait()

            pltpu.sync_copy(x_hbm.at[pl.ds(base * D, RB * D)], xr)

            @plsc.parallel_loop(0, FLAT // LANES, unroll=8)
            def _(o):
                off = o * LANES
                iv = idx_v[pl.ds(off, LANES)]
                for r in range(RB):
                    ov[pl.ds(r * FLAT + off, LANES)] = (
                        plsc.load_gather(xr, [iv + r * D]))

            pltpu.async_copy(ov, dst, sem)
        return 0

    lax.fori_loop(0, RPW // (2 * RB), pair, 0)
    for half in range(2):
        xr, ov, sem = bufs[half]
        pltpu.make_async_copy(ov, out_hbm.at[pl.ds(row0 * FLAT, RB * FLAT)], sem).wait()


_sc_gather_cached = None


def _sc_gather(x_flat, idx_flat):
    global _sc_gather_cached
    if _sc_gather_cached is None:
        _sc_gather_cached = pl.kernel(
            _sc_gather_body,
            out_type=jax.ShapeDtypeStruct((ROWS * FLAT,), jnp.float32),
            mesh=plsc.VectorSubcoreMesh(
                core_axis_name="c", subcore_axis_name="s",
                num_cores=2, num_subcores=16,
            ),
            compiler_params=pltpu.CompilerParams(needs_layout_passes=False),
            scratch_types=[
                pltpu.VMEM((FLAT,), jnp.int32),
                pltpu.VMEM((RB * D,), jnp.float32),
                pltpu.VMEM((RB * D,), jnp.float32),
                pltpu.VMEM((RB * FLAT,), jnp.float32),
                pltpu.VMEM((RB * FLAT,), jnp.float32),
                pltpu.SemaphoreType.DMA,
                pltpu.SemaphoreType.DMA,
            ],
        )
    return _sc_gather_cached(x_flat, idx_flat)


IG = 8


def _tc_gather_body(x_ref, sel_ref, out_ref):
    x2 = x_ref[0]
    hi = x2.astype(jnp.bfloat16)
    lo = (x2 - hi.astype(jnp.float32)).astype(jnp.bfloat16)
    selrow = sel_ref[0]
    jidx = lax.broadcasted_iota(jnp.int32, (D, IG * K), 0)
    p = (jidx == selrow[None, :]).astype(jnp.bfloat16)
    acc = (jnp.dot(hi, p, preferred_element_type=jnp.float32)
           + jnp.dot(lo, p, preferred_element_type=jnp.float32))
    out_ref[...] = acc.reshape(1, S, IG, K)


_tc_gather = pl.pallas_call(
    _tc_gather_body,
    grid=(B, D // IG),
    in_specs=[
        pl.BlockSpec((1, S, D), lambda b, g: (b, 0, 0)),
        pl.BlockSpec((1, IG * K), lambda b, g: (0, g)),
    ],
    out_specs=pl.BlockSpec((1, S, IG, K), lambda b, g: (b, 0, g, 0)),
    out_shape=jax.ShapeDtypeStruct((B, S, D, K), jnp.float32),
)


def kernel(x, importance_scores, W, b):
    fw, sel = _gate_topk(W.T, b.reshape(1, D), importance_scores.reshape(1, D))
    out = _tc_gather(x, sel.reshape(1, FLAT))
    return out, fw


# speedup vs baseline: 4.2345x; 3.9754x over previous
import jax
import jax.numpy as jnp
from jax import lax
from jax.experimental import pallas as pl
from jax.experimental.pallas import tpu as pltpu
from jax.experimental.pallas import tpu_sc as plsc

D = 256
K = 64
B = 8
S = 256
ROWS = B * S
FLAT = D * K
NW = 32
RPW = ROWS // NW
RB = 2
LANES = 16


def _gate_topk_body(wt_ref, b_ref, imp_ref, fw_ref, sel_ref, selt_ref):
    imp = imp_ref[...]
    m = jnp.max(imp, axis=1, keepdims=True)
    e = jnp.exp(imp - m)
    wts = e / jnp.sum(e, axis=1, keepdims=True)
    wt = wt_ref[...].astype(jnp.bfloat16).astype(jnp.float32)
    g = jax.nn.sigmoid(wt + b_ref[...])
    fw = wts * g
    fw_ref[...] = fw

    jidx = lax.broadcasted_iota(jnp.int32, (D, D), 1)
    work = fw
    for r in range(K):
        mx = jnp.max(work, axis=1, keepdims=True)
        cand = jnp.where(work == mx, jidx, D)
        j = jnp.min(cand, axis=1, keepdims=True)
        sel_ref[:, r:r + 1] = j
        work = jnp.where(jidx == j, -jnp.inf, work)
    selt_ref[...] = sel_ref[...].T


_gate_topk = pl.pallas_call(
    _gate_topk_body,
    out_shape=(
        jax.ShapeDtypeStruct((D, D), jnp.float32),
        jax.ShapeDtypeStruct((D, K), jnp.int32),
        jax.ShapeDtypeStruct((K, D), jnp.int32),
    ),
)


def _sc_gather_body(x_hbm, idx_hbm, out_hbm,
                    idx_v, xr0, xr1, ov0, ov1, sem0, sem1):
    cid = lax.axis_index("c")
    sid = lax.axis_index("s")
    wid = sid * 2 + cid
    row0 = wid * RPW
    pltpu.sync_copy(idx_hbm, idx_v)

    bufs = ((xr0, ov0, sem0), (xr1, ov1, sem1))

    def pair(i, _):
        for half in range(2):
            xr, ov, sem = bufs[half]
            rb = i * 2 + half
            base = row0 + rb * RB

            dst = out_hbm.at[pl.ds(base * FLAT, RB * FLAT)]

            @pl.when(i > 0)
            def _():
                pltpu.make_async_copy(ov, dst, sem).wait()

            pltpu.sync_copy(x_hbm.at[pl.ds(base * D, RB * D)], xr)

            @plsc.parallel_loop(0, FLAT // LANES, unroll=8)
            def _(o):
                off = o * LANES
                iv = idx_v[pl.ds(off, LANES)]
                for r in range(RB):
                    ov[pl.ds(r * FLAT + off, LANES)] = (
                        plsc.load_gather(xr, [iv + r * D]))

            pltpu.async_copy(ov, dst, sem)
        return 0

    lax.fori_loop(0, RPW // (2 * RB), pair, 0)
    for half in range(2):
        xr, ov, sem = bufs[half]
        pltpu.make_async_copy(ov, out_hbm.at[pl.ds(row0 * FLAT, RB * FLAT)], sem).wait()


_sc_gather_cached = None


def _sc_gather(x_flat, idx_flat):
    global _sc_gather_cached
    if _sc_gather_cached is None:
        _sc_gather_cached = pl.kernel(
            _sc_gather_body,
            out_type=jax.ShapeDtypeStruct((ROWS * FLAT,), jnp.float32),
            mesh=plsc.VectorSubcoreMesh(
                core_axis_name="c", subcore_axis_name="s",
                num_cores=2, num_subcores=16,
            ),
            compiler_params=pltpu.CompilerParams(needs_layout_passes=False),
            scratch_types=[
                pltpu.VMEM((FLAT,), jnp.int32),
                pltpu.VMEM((RB * D,), jnp.float32),
                pltpu.VMEM((RB * D,), jnp.float32),
                pltpu.VMEM((RB * FLAT,), jnp.float32),
                pltpu.VMEM((RB * FLAT,), jnp.float32),
                pltpu.SemaphoreType.DMA,
                pltpu.SemaphoreType.DMA,
            ],
        )
    return _sc_gather_cached(x_flat, idx_flat)


KG = 8


def _tc_gather_body(x_ref, selt_ref, out_ref):
    x2 = x_ref[0]
    hi = x2.astype(jnp.bfloat16)
    lo = (x2 - hi.astype(jnp.float32)).astype(jnp.bfloat16)
    jidx = lax.broadcasted_iota(jnp.int32, (D, D), 0)
    for kk in range(KG):
        t = selt_ref[kk:kk + 1, :]
        p = (jidx == t).astype(jnp.bfloat16)
        acc = (jnp.dot(hi, p, preferred_element_type=jnp.float32)
               + jnp.dot(lo, p, preferred_element_type=jnp.float32))
        out_ref[0, :, kk, :] = acc


_tc_gather = pl.pallas_call(
    _tc_gather_body,
    grid=(B, K // KG),
    in_specs=[
        pl.BlockSpec((1, S, D), lambda b, g: (b, 0, 0)),
        pl.BlockSpec((KG, D), lambda b, g: (g, 0)),
    ],
    out_specs=pl.BlockSpec((1, S, KG, D), lambda b, g: (b, 0, g, 0)),
    out_shape=jax.ShapeDtypeStruct((B, S, K, D), jnp.float32),
)


def kernel(x, importance_scores, W, b):
    fw, sel, selt = _gate_topk(
        W.T, b.reshape(1, D), importance_scores.reshape(1, D))
    out_t = _tc_gather(x, selt)
    return jnp.transpose(out_t, (0, 1, 3, 2)), fw
